# baseline (speedup 1.0000x reference)
import functools

import jax
import jax.numpy as jnp
from jax import lax
from jax.experimental import pallas as pl
from jax.experimental.pallas import tpu as pltpu
from jax.experimental.pallas import tpu_sc as plsc

LANES = 16
NBUF = 2
SLABW = 512
SHIFT = 9


def _dsa(start, size, align):
    return pl.ds(pl.multiple_of(start, align), size)


def _splat(x):
    return jnp.broadcast_to(jnp.asarray(x, jnp.int32), (LANES,))


@functools.lru_cache(maxsize=None)
def _build(batch, n_fields, v_rows, d, n_cores):
    total = batch * n_fields
    n_vec = batch // LANES
    hist_n = 512
    d_vecs = d // LANES
    blocks = batch // 128

    mesh = plsc.VectorSubcoreMesh(core_axis_name="c", subcore_axis_name="s")

    @functools.partial(
        pl.kernel,
        mesh=mesh,
        out_type=jax.ShapeDtypeStruct((total, 128), jnp.float32),
        scratch_types=[
            pltpu.VMEM((batch,), jnp.int32),
            pltpu.VMEM((batch,), jnp.int32),
            pltpu.VMEM((batch,), jnp.int32),
            pltpu.VMEM((batch,), jnp.int32),
            pltpu.VMEM((batch,), jnp.int32),
            pltpu.VMEM((hist_n,), jnp.int32),
            pltpu.VMEM((hist_n,), jnp.int32),
            pltpu.VMEM((blocks, 128), jnp.int32),
            pltpu.VMEM((n_fields,), jnp.int32),
            pltpu.VMEM((n_fields * d,), jnp.float32),
            pltpu.VMEM((NBUF, d, SLABW), jnp.float32),
            pltpu.VMEM((2, 128, 128), jnp.float32),
            pltpu.SemaphoreType.DMA,
            pltpu.SemaphoreType.DMA,
            pltpu.SemaphoreType.DMA,
            pltpu.SemaphoreType.DMA,
        ],
        compiler_params=pltpu.CompilerParams(needs_layout_passes=False),
    )
    def emb_kernel(xt_hbm, off_hbm, tbl_hbm, tail_hbm, bias_hbm, stage_hbm,
                   g_v, s_v, rt_v, srt_g, srt_b, pref_v, run_v, oid_v, off_v,
                   bias_v, slab_v, outb_v, sem0, sem1, ssem0, ssem1):
        sems = [sem0, sem1]
        ssems = [ssem0, ssem1]
        cid = lax.axis_index("c")
        sid = lax.axis_index("s")
        wid = sid * n_cores + cid

        @pl.when(wid < n_fields)
        def _worker():
            iota = lax.iota(jnp.int32, LANES)

            pltpu.sync_copy(xt_hbm.at[wid], g_v)
            pltpu.sync_copy(off_hbm, off_v)
            pltpu.sync_copy(bias_hbm, bias_v)

            offv = plsc.load_gather(off_v, [_splat(wid)])
            off_s = jnp.max(offv)
            nxtv = plsc.load_gather(
                off_v, [_splat(jnp.minimum(wid + 1, n_fields - 1))])
            end_g = jnp.where(wid + 1 < n_fields, jnp.max(nxtv),
                              jnp.int32(v_rows))
            first_slab = lax.shift_right_logical(off_s, SHIFT)
            last_slab = lax.shift_right_logical(end_g - 1, SHIFT)
            nslab = last_slab - first_slab + 1
            has_tail = (last_slab + 1) * SLABW > v_rows

            bvs = [plsc.load_gather(bias_v, [_splat(wid * d) + iota + jb * LANES])
                   for jb in range(d_vecs)]

            def prep(i, c):
                sl = _dsa(i * LANES, LANES, 8)
                g = g_v[sl] + offv
                g_v[sl] = g
                s_v[sl] = lax.shift_right_logical(g, SHIFT) - first_slab
                return c
            lax.fori_loop(0, n_vec, prep, 0)

            for k in range(hist_n // LANES):
                pref_v[pl.ds(k * LANES, LANES)] = jnp.zeros((LANES,), jnp.int32)

            def hist(i, c):
                sv = jnp.clip(s_v[_dsa(i * LANES, LANES, 8)], 0, hist_n - 2)
                r, m = plsc.scan_count(sv)
                base = plsc.load_gather(pref_v, [sv])
                plsc.store_scatter(pref_v, [sv], base + r, mask=m)
                return c
            lax.fori_loop(0, n_vec, hist, 0)

            carry = jnp.int32(0)
            for k in range(hist_n // LANES):
                sl = pl.ds(k * LANES, LANES)
                h = pref_v[sl]
                inc = plsc.cumsum(h)
                excl = inc - h + jnp.broadcast_to(carry, (LANES,))
                pref_v[sl] = excl
                run_v[sl] = excl
                carry = carry + jnp.max(inc)

            def rank_pass(i, c):
                sl = _dsa(i * LANES, LANES, 8)
                sv = jnp.clip(s_v[sl], 0, hist_n - 2)
                r, m = plsc.scan_count(sv)
                base = plsc.load_gather(run_v, [sv])
                slot = jnp.clip(base + r - 1, 0, batch - 1)
                plsc.store_scatter(srt_g, [slot], g_v[sl])
                plsc.store_scatter(srt_b, [slot], _splat(i * LANES) + iota)
                plsc.store_scatter(run_v, [sv], base + r, mask=m)
                return c
            lax.fori_loop(0, n_vec, rank_pass, 0)

            def post(i, c):
                sl = _dsa(i * LANES, LANES, 8)
                s_v[sl] = jnp.bitwise_and(srt_g[sl], SLABW - 1)
                return c
            lax.fori_loop(0, n_vec, post, 0)

            def oid(j, c):
                for k in range(128 // LANES):
                    bv = srt_b[_dsa(j * 128 + k * LANES, LANES, 8)]
                    oid_v[j, pl.ds(k * LANES, LANES)] = jnp.clip(
                        bv * n_fields + _splat(wid), 0, total - 1)
                return c
            lax.fori_loop(0, blocks, oid, 0)

            def issue(s_idx, p):
                @pl.when(s_idx < nslab)
                def _():
                    is_tail = jnp.logical_and(has_tail, s_idx == nslab - 1)

                    @pl.when(is_tail)
                    def _():
                        pltpu.async_copy(tail_hbm, slab_v.at[p], sems[p])

                    @pl.when(jnp.logical_not(is_tail))
                    def _():
                        col0 = (first_slab + s_idx) * SLABW
                        pltpu.async_copy(
                            tbl_hbm.at[:, _dsa(col0, SLABW, 128)],
                            slab_v.at[p], sems[p])

            for p in range(NBUF):
                issue(jnp.int32(p), p)

            def sweep(t, c):
                for p in range(NBUF):
                    s_idx = t * NBUF + p

                    @pl.when(s_idx < nslab)
                    def _(p=p, s_idx=s_idx):
                        pltpu.make_async_copy(
                            tbl_hbm.at[:, _dsa(0, SLABW, 128)],
                            slab_v.at[p], sems[p]).wait()
                        hi = jnp.minimum(
                            jnp.max(plsc.load_gather(pref_v, [_splat(s_idx + 1)])),
                            jnp.int32(batch))
                        lo = jnp.minimum(
                            jnp.max(plsc.load_gather(pref_v, [_splat(s_idx)])), hi)

                        def occ(q):
                            lanev = plsc.load_gather(
                                s_v, [jnp.clip(_splat(q), 0, batch - 1)])
                            blk = lax.shift_right_logical(q, 7)
                            par = jnp.bitwise_and(blk, 1)
                            row = jnp.bitwise_and(q, 127)

                            for sp in range(2):
                                @pl.when(jnp.logical_and(
                                    jnp.logical_and(row == 0, blk >= 2),
                                    par == sp))
                                def _(sp=sp):
                                    pltpu.make_async_copy(
                                        outb_v.at[sp],
                                        stage_hbm.at[oid_v.at[0]],
                                        ssems[sp]).wait()

                            for jb in range(d_vecs):
                                val = plsc.load_gather(
                                    slab_v.at[p], [iota + jb * LANES, lanev])
                                outb_v[par, row, pl.ds(jb * LANES, LANES)] = \
                                    val + bvs[jb]

                            for sp in range(2):
                                @pl.when(jnp.logical_and(row == 127, par == sp))
                                def _(sp=sp):
                                    pltpu.async_copy(
                                        outb_v.at[sp],
                                        stage_hbm.at[oid_v.at[blk]],
                                        ssems[sp])
                            return q + 1
                        lax.while_loop(lambda q: q < hi, occ, lo)
                        issue(s_idx + NBUF, p)
                return c
            tmax = (nslab + NBUF - 1) // NBUF
            lax.fori_loop(0, tmax, sweep, 0)

            for sp in range(2):
                pltpu.make_async_copy(
                    outb_v.at[sp], stage_hbm.at[oid_v.at[0]], ssems[sp]).wait()

    return emb_kernel


def kernel(x, table, bias, offsets):
    batch, n_fields = x.shape
    v_rows, d = table.shape

    info = plsc.get_sparse_core_info()

    xt = x.T.astype(jnp.int32)
    tbl_t = table.T
    tail_w = v_rows % SLABW
    tail_base = v_rows - tail_w
    if tail_w:
        tail = jnp.concatenate(
            [table[tail_base:].T,
             jnp.zeros((d, SLABW - tail_w), jnp.float32)], axis=1)
    else:
        tail = jnp.zeros((d, SLABW), jnp.float32)

    staging = _build(batch, n_fields, v_rows, d, info.num_cores)(
        xt, offsets.astype(jnp.int32), tbl_t, tail, bias.reshape(-1))
    return staging[:, :d].reshape(batch, n_fields, d)


# speedup vs baseline: 1.1391x; 1.1391x over previous
import functools

import jax
import jax.numpy as jnp
from jax import lax
from jax.experimental import pallas as pl
from jax.experimental.pallas import tpu as pltpu
from jax.experimental.pallas import tpu_sc as plsc

LANES = 16
NBUF = 4
SLABW = 256
SHIFT = 8


def _dsa(start, size, align):
    return pl.ds(pl.multiple_of(start, align), size)


def _splat(x):
    return jnp.broadcast_to(jnp.asarray(x, jnp.int32), (LANES,))


@functools.lru_cache(maxsize=None)
def _build(batch, n_fields, v_rows, d, n_cores):
    total = batch * n_fields
    n_vec = batch // LANES
    hist_n = 512
    d_vecs = d // LANES
    blocks = batch // 128

    mesh = plsc.VectorSubcoreMesh(core_axis_name="c", subcore_axis_name="s")

    @functools.partial(
        pl.kernel,
        mesh=mesh,
        out_type=jax.ShapeDtypeStruct((total, 128), jnp.float32),
        scratch_types=[
            pltpu.VMEM((batch,), jnp.int32),
            pltpu.VMEM((batch,), jnp.int32),
            pltpu.VMEM((batch,), jnp.int32),
            pltpu.VMEM((batch,), jnp.int32),
            pltpu.VMEM((batch,), jnp.int32),
            pltpu.VMEM((hist_n,), jnp.int32),
            pltpu.VMEM((hist_n,), jnp.int32),
            pltpu.VMEM((blocks, 128), jnp.int32),
            pltpu.VMEM((n_fields,), jnp.int32),
            pltpu.VMEM((n_fields * d,), jnp.float32),
            pltpu.VMEM((NBUF, d, SLABW), jnp.float32),
            pltpu.VMEM((2, 128, 128), jnp.float32),
            pltpu.SemaphoreType.DMA,
            pltpu.SemaphoreType.DMA,
            pltpu.SemaphoreType.DMA,
            pltpu.SemaphoreType.DMA,
            pltpu.SemaphoreType.DMA,
            pltpu.SemaphoreType.DMA,
        ],
        compiler_params=pltpu.CompilerParams(needs_layout_passes=False),
    )
    def emb_kernel(xt_hbm, off_hbm, tbl_hbm, tail_hbm, bias_hbm, stage_hbm,
                   g_v, s_v, rt_v, srt_g, srt_b, pref_v, run_v, oid_v, off_v,
                   bias_v, slab_v, outb_v, sem0, sem1, sem2, sem3, ssem0,
                   ssem1):
        sems = [sem0, sem1, sem2, sem3]
        ssems = [ssem0, ssem1]
        cid = lax.axis_index("c")
        sid = lax.axis_index("s")
        wid = sid * n_cores + cid

        @pl.when(wid < n_fields)
        def _worker():
            iota = lax.iota(jnp.int32, LANES)

            pltpu.sync_copy(xt_hbm.at[wid], g_v)
            pltpu.sync_copy(off_hbm, off_v)
            pltpu.sync_copy(bias_hbm, bias_v)

            offv = plsc.load_gather(off_v, [_splat(wid)])
            off_s = jnp.max(offv)
            nxtv = plsc.load_gather(
                off_v, [_splat(jnp.minimum(wid + 1, n_fields - 1))])
            end_g = jnp.where(wid + 1 < n_fields, jnp.max(nxtv),
                              jnp.int32(v_rows))
            first_slab = lax.shift_right_logical(off_s, SHIFT)
            last_slab = lax.shift_right_logical(end_g - 1, SHIFT)
            nslab = last_slab - first_slab + 1
            has_tail = (last_slab + 1) * SLABW > v_rows

            bvs = [plsc.load_gather(bias_v, [_splat(wid * d) + iota + jb * LANES])
                   for jb in range(d_vecs)]

            def prep(i, c):
                sl = _dsa(i * LANES, LANES, 8)
                g = g_v[sl] + offv
                g_v[sl] = g
                s_v[sl] = lax.shift_right_logical(g, SHIFT) - first_slab
                return c
            lax.fori_loop(0, n_vec, prep, 0)

            for k in range(hist_n // LANES):
                pref_v[pl.ds(k * LANES, LANES)] = jnp.zeros((LANES,), jnp.int32)

            def hist(i, c):
                sv = jnp.clip(s_v[_dsa(i * LANES, LANES, 8)], 0, hist_n - 2)
                r, m = plsc.scan_count(sv)
                base = plsc.load_gather(pref_v, [sv])
                plsc.store_scatter(pref_v, [sv], base + r, mask=m)
                return c
            lax.fori_loop(0, n_vec, hist, 0)

            carry = jnp.int32(0)
            for k in range(hist_n // LANES):
                sl = pl.ds(k * LANES, LANES)
                h = pref_v[sl]
                inc = plsc.cumsum(h)
                excl = inc - h + jnp.broadcast_to(carry, (LANES,))
                pref_v[sl] = excl
                run_v[sl] = excl
                carry = carry + jnp.max(inc)

            def rank_pass(i, c):
                sl = _dsa(i * LANES, LANES, 8)
                sv = jnp.clip(s_v[sl], 0, hist_n - 2)
                r, m = plsc.scan_count(sv)
                base = plsc.load_gather(run_v, [sv])
                slot = jnp.clip(base + r - 1, 0, batch - 1)
                plsc.store_scatter(srt_g, [slot], g_v[sl])
                plsc.store_scatter(srt_b, [slot], _splat(i * LANES) + iota)
                plsc.store_scatter(run_v, [sv], base + r, mask=m)
                return c
            lax.fori_loop(0, n_vec, rank_pass, 0)

            def post(i, c):
                sl = _dsa(i * LANES, LANES, 8)
                s_v[sl] = jnp.bitwise_and(srt_g[sl], SLABW - 1)
                return c
            lax.fori_loop(0, n_vec, post, 0)

            def oid(j, c):
                for k in range(128 // LANES):
                    bv = srt_b[_dsa(j * 128 + k * LANES, LANES, 8)]
                    oid_v[j, pl.ds(k * LANES, LANES)] = jnp.clip(
                        bv * n_fields + _splat(wid), 0, total - 1)
                return c
            lax.fori_loop(0, blocks, oid, 0)

            def issue(s_idx, p):
                @pl.when(s_idx < nslab)
                def _():
                    is_tail = jnp.logical_and(has_tail, s_idx == nslab - 1)

                    @pl.when(is_tail)
                    def _():
                        pltpu.async_copy(tail_hbm, slab_v.at[p], sems[p])

                    @pl.when(jnp.logical_not(is_tail))
                    def _():
                        col0 = (first_slab + s_idx) * SLABW
                        pltpu.async_copy(
                            tbl_hbm.at[:, _dsa(col0, SLABW, 128)],
                            slab_v.at[p], sems[p])

            for p in range(NBUF):
                issue(jnp.int32(p), p)

            def sweep(t, c):
                for p in range(NBUF):
                    s_idx = t * NBUF + p

                    @pl.when(s_idx < nslab)
                    def _(p=p, s_idx=s_idx):
                        pltpu.make_async_copy(
                            tbl_hbm.at[:, _dsa(0, SLABW, 128)],
                            slab_v.at[p], sems[p]).wait()
                        hi = jnp.minimum(
                            jnp.max(plsc.load_gather(pref_v, [_splat(s_idx + 1)])),
                            jnp.int32(batch))
                        lo = jnp.minimum(
                            jnp.max(plsc.load_gather(pref_v, [_splat(s_idx)])), hi)

                        def occ(q):
                            lanev = plsc.load_gather(
                                s_v, [jnp.clip(_splat(q), 0, batch - 1)])
                            blk = lax.shift_right_logical(q, 7)
                            par = jnp.bitwise_and(blk, 1)
                            row = jnp.bitwise_and(q, 127)

                            for sp in range(2):
                                @pl.when(jnp.logical_and(
                                    jnp.logical_and(row == 0, blk >= 2),
                                    par == sp))
                                def _(sp=sp):
                                    pltpu.make_async_copy(
                                        outb_v.at[sp],
                                        stage_hbm.at[oid_v.at[0]],
                                        ssems[sp]).wait()

                            for jb in range(d_vecs):
                                val = plsc.load_gather(
                                    slab_v.at[p], [iota + jb * LANES, lanev])
                                outb_v[par, row, pl.ds(jb * LANES, LANES)] = \
                                    val + bvs[jb]

                            for sp in range(2):
                                @pl.when(jnp.logical_and(row == 127, par == sp))
                                def _(sp=sp):
                                    pltpu.async_copy(
                                        outb_v.at[sp],
                                        stage_hbm.at[oid_v.at[blk]],
                                        ssems[sp])
                            return q + 1
                        lax.while_loop(lambda q: q < hi, occ, lo)
                        issue(s_idx + NBUF, p)
                return c
            tmax = (nslab + NBUF - 1) // NBUF
            lax.fori_loop(0, tmax, sweep, 0)

            for sp in range(2):
                pltpu.make_async_copy(
                    outb_v.at[sp], stage_hbm.at[oid_v.at[0]], ssems[sp]).wait()

    return emb_kernel


def kernel(x, table, bias, offsets):
    batch, n_fields = x.shape
    v_rows, d = table.shape

    info = plsc.get_sparse_core_info()

    xt = x.T.astype(jnp.int32)
    tbl_t = table.T
    tail_w = v_rows % SLABW
    tail_base = v_rows - tail_w
    if tail_w:
        tail = jnp.concatenate(
            [table[tail_base:].T,
             jnp.zeros((d, SLABW - tail_w), jnp.float32)], axis=1)
    else:
        tail = jnp.zeros((d, SLABW), jnp.float32)

    staging = _build(batch, n_fields, v_rows, d, info.num_cores)(
        xt, offsets.astype(jnp.int32), tbl_t, tail, bias.reshape(-1))
    return staging[:, :d].reshape(batch, n_fields, d)
